# Initial kernel scaffold; baseline (speedup 1.0000x reference)
import jax
import jax.numpy as jnp
from jax.experimental import pallas as pl


def kernel(x, sa1_params, sa2_params, sa3_params, head_params):
    raise NotImplementedError("write your pallas kernel here")


# speedup vs baseline: 1.0001x; 1.0001x over previous
import jax
import jax.numpy as jnp
from jax.experimental import pallas as pl


def _square_distance(src, dst):
    return (jnp.sum(src ** 2, -1)[:, :, None] + jnp.sum(dst ** 2, -1)[:, None, :]
            - 2.0 * jnp.einsum('bnc,bmc->bnm', src, dst))


def _index_points(points, idx):
    B = points.shape[0]
    batch = jnp.arange(B).reshape((B,) + (1,) * (idx.ndim - 1))
    return points[batch, idx]


def _fps(xyz, npoint):
    B, N, _ = xyz.shape
    centroids = []
    distance = jnp.full((B, N), 1e10, dtype=jnp.float32)
    farthest = jnp.zeros((B,), dtype=jnp.int32)
    for _ in range(npoint):
        centroids.append(farthest)
        centroid = xyz[jnp.arange(B), farthest][:, None, :]
        dist = jnp.sum((xyz - centroid) ** 2, axis=-1)
        distance = jnp.minimum(distance, dist)
        farthest = jnp.argmax(distance, axis=-1).astype(jnp.int32)
    return jnp.stack(centroids, axis=1)


def _query_ball(radius, k, xyz, new_xyz):
    B, N, _ = xyz.shape
    S = new_xyz.shape[1]
    sqrdists = _square_distance(new_xyz, xyz)
    group_idx = jnp.broadcast_to(jnp.arange(N, dtype=jnp.int32), (B, S, N))
    group_idx = jnp.where(sqrdists > radius * radius, N, group_idx)
    group_idx = jnp.sort(group_idx, axis=-1)[:, :, :k]
    group_first = group_idx[:, :, :1]
    group_idx = jnp.where(group_idx == N, group_first, group_idx)
    return group_idx


def _conv_mlp(feats, params):
    for W, b in params:
        feats = jnp.einsum('bskc,oc->bsko', feats, W) + b
        feats = jax.nn.relu(feats)
    return feats


def _sa_msg(xyz, feats, n_group, k_list, r_list, params_list):
    fps_idx = _fps(xyz, n_group)
    new_xyz = _index_points(xyz, fps_idx)
    outs = []
    for k, r, params in zip(k_list, r_list, params_list):
        idx = _query_ball(r, k, xyz, new_xyz)
        grouped_xyz = _index_points(xyz, idx) - new_xyz[:, :, None, :]
        grouped = jnp.concatenate([grouped_xyz, _index_points(feats, idx)], axis=-1)
        h = _conv_mlp(grouped, params)
        outs.append(jnp.max(h, axis=2))
    return new_xyz, jnp.concatenate(outs, axis=-1)


def _head_kernel(feat_ref, w1, b1, w2, b2, w3, b3, out_ref):
    f = feat_ref[...]
    f = jax.nn.relu(f @ w1[...].T + b1[...])
    f = jax.nn.relu(f @ w2[...].T + b2[...])
    f = f @ w3[...].T + b3[...]
    out_ref[...] = jax.nn.log_softmax(f, axis=-1)


def kernel(x, sa1_params, sa2_params, sa3_params, head_params):
    pts = jnp.transpose(x, (0, 2, 1))
    xyz1, f1 = _sa_msg(pts, pts, 512, [16, 32, 128], [0.1, 0.2, 0.4], sa1_params)
    xyz2, f2 = _sa_msg(xyz1, f1, 128, [32, 64, 128], [0.2, 0.4, 0.8], sa2_params)
    grouped = jnp.concatenate([xyz2, f2], axis=-1)[:, None, :, :]
    f3 = jnp.max(_conv_mlp(grouped, sa3_params), axis=2)
    feat = f3.reshape(-1, 1024)
    (w1, b1), (w2, b2), (w3, b3) = head_params
    out = pl.pallas_call(
        _head_kernel,
        out_shape=jax.ShapeDtypeStruct((feat.shape[0], w3.shape[0]), jnp.float32),
    )(feat, w1, b1, w2, b2, w3, b3)
    return out


# speedup vs baseline: 14.2359x; 14.2343x over previous
import functools

import jax
import jax.numpy as jnp
from jax import lax
from jax.experimental import pallas as pl
from jax.experimental.pallas import tpu as pltpu
from jax.experimental.pallas import tpu_sc as plsc


def _fps_body(npoint, xs_ref, ys_ref, zs_ref, ox_ref, oy_ref, oz_ref,
              opc_ref, dist_ref):
    B, N = xs_ref.shape
    xs = xs_ref[...]
    ys = ys_ref[...]
    zs = zs_ref[...]
    lanes = lax.broadcasted_iota(jnp.int32, (B, N), 1)
    out_lanes = lax.broadcasted_iota(jnp.int32, (B, npoint), 1)
    dist_ref[...] = jnp.full((B, N), 1e10, jnp.float32)

    def step(i, far):
        onehot = lanes == far
        cx = jnp.sum(jnp.where(onehot, xs, 0.0), axis=1, keepdims=True)
        cy = jnp.sum(jnp.where(onehot, ys, 0.0), axis=1, keepdims=True)
        cz = jnp.sum(jnp.where(onehot, zs, 0.0), axis=1, keepdims=True)
        dx = xs - cx
        dy = ys - cy
        dz = zs - cz
        d = (dx * dx + dy * dy) + dz * dz
        dist = jnp.minimum(dist_ref[...], d)
        dist_ref[...] = dist
        m = jnp.max(dist, axis=1, keepdims=True)
        far_new = jnp.min(jnp.where(dist == m, lanes, N), axis=1,
                          keepdims=True).astype(jnp.int32)
        sel = out_lanes == i
        ox_ref[...] = jnp.where(sel, cx, ox_ref[...])
        oy_ref[...] = jnp.where(sel, cy, oy_ref[...])
        oz_ref[...] = jnp.where(sel, cz, oz_ref[...])
        opc_ref[...] = jnp.where(sel, (cx * cx + cy * cy) + cz * cz,
                                 opc_ref[...])
        return far_new

    lax.fori_loop(0, npoint, step, jnp.zeros((B, 1), jnp.int32))


def _fps(xs, ys, zs, npoint):
    B, N = xs.shape
    out = jax.ShapeDtypeStruct((B, npoint), jnp.float32)
    return pl.pallas_call(
        functools.partial(_fps_body, npoint),
        out_shape=[out, out, out, out],
        scratch_shapes=[pltpu.VMEM((B, N), jnp.float32)],
    )(xs, ys, zs)


_SC_MESH = dict(core_axis_name="c", subcore_axis_name="s")


def _bq1_body(N, S, ks, r2s, xs_h, ys_h, zs_h, cx_h, cy_h, cz_h, pc_h,
              g1_h, g2_h, g3_h,
              xs_v, ys_v, zs_v, px_v, cxs_v, cys_v, czs_v, pcs_v,
              i1_v, i2_v, i3_v, row_v):
    k1, k2, k3 = ks
    r21, r22, r23 = r2s
    w = lax.axis_index("s") * 2 + lax.axis_index("c")
    b = w // 4
    q = w % 4
    S_sub = S // 4
    s0 = q * S_sub
    pltpu.sync_copy(xs_h.at[pl.ds(b * N, N)], xs_v)
    pltpu.sync_copy(ys_h.at[pl.ds(b * N, N)], ys_v)
    pltpu.sync_copy(zs_h.at[pl.ds(b * N, N)], zs_v)
    pltpu.sync_copy(cx_h.at[pl.ds(b * S + s0, S_sub)], cxs_v)
    pltpu.sync_copy(cy_h.at[pl.ds(b * S + s0, S_sub)], cys_v)
    pltpu.sync_copy(cz_h.at[pl.ds(b * S + s0, S_sub)], czs_v)
    pltpu.sync_copy(pc_h.at[pl.ds(b * S + s0, S_sub)], pcs_v)
    iota = lax.iota(jnp.int32, 16)

    def px_step(t, _):
        sl = pl.ds(t * 16, 16)
        xv = xs_v[sl]
        yv = ys_v[sl]
        zv = zs_v[sl]
        px_v[sl] = (xv * xv + yv * yv) + zv * zv
        return 0

    lax.fori_loop(0, N // 16, px_step, 0)

    def scal(ref, i):
        chv = ref[pl.ds((i // 16) * 16, 16)]
        return jnp.sum(jnp.where(iota == i % 16, chv,
                                 jnp.zeros((16,), chv.dtype)))

    def row_fn(rl, _):
        cx_s = scal(cxs_v, rl)
        cy_s = scal(cys_v, rl)
        cz_s = scal(czs_v, rl)
        pc_s = scal(pcs_v, rl)

        def chunk(t, cnts):
            c1, c2, c3 = cnts
            sl = pl.ds(t * 16, 16)
            xv = xs_v[sl]
            yv = ys_v[sl]
            zv = zs_v[sl]
            pxv = px_v[sl]
            dot = (cx_s * xv + cy_s * yv) + cz_s * zv
            d = (pc_s + pxv) - 2.0 * dot
            nvec = t * 16 + iota

            def one(r2, kk, buf, cnt):
                m = d <= r2
                ci = jnp.cumsum(m.astype(jnp.int32))
                pos = cnt + ci - 1
                wr = m & (pos < kk)
                plsc.store_scatter(buf, [jnp.maximum(pos, 0)], nvec, mask=wr)
                return cnt + jnp.sum(m.astype(jnp.int32))

            c1 = one(r21, k1, i1_v, c1)
            c2 = one(r22, k2, i2_v, c2)
            c3 = one(r23, k3, i3_v, c3)
            return (c1, c2, c3)

        zero = jnp.array(0, jnp.int32)
        c1, c2, c3 = lax.fori_loop(0, N // 16, chunk, (zero, zero, zero))
        sg = s0 + rl

        def emit(kk, cnt, buf, out_h):
            first = scal(buf, 0)

            def fill(ch, _):
                slot = ch * 16 + iota
                cur = buf[pl.ds(ch * 16, 16)]
                vals = jnp.where(slot < cnt, cur, first)
                gx = plsc.load_gather(xs_v, [vals])
                gy = plsc.load_gather(ys_v, [vals])
                gz = plsc.load_gather(zs_v, [vals])
                base = slot * 6
                plsc.store_scatter(row_v, [base], gx - cx_s)
                plsc.store_scatter(row_v, [base + 1], gy - cy_s)
                plsc.store_scatter(row_v, [base + 2], gz - cz_s)
                plsc.store_scatter(row_v, [base + 3], gx)
                plsc.store_scatter(row_v, [base + 4], gy)
                plsc.store_scatter(row_v, [base + 5], gz)
                return 0

            lax.fori_loop(0, kk // 16, fill, 0)
            pltpu.sync_copy(row_v.at[pl.ds(0, kk * 6)],
                            out_h.at[pl.ds((b * S + sg) * kk * 6, kk * 6)])

        emit(k1, c1, i1_v, g1_h)
        emit(k2, c2, i2_v, g2_h)
        emit(k3, c3, i3_v, g3_h)
        return 0

    lax.fori_loop(0, S_sub, row_fn, 0)


def _ball_group1(xs, ys, zs, cx, cy, cz, pc, ks, radii):
    B, N = xs.shape
    S = cx.shape[1]
    r2s = tuple(float(r) * float(r) for r in radii)
    kern = pl.kernel(
        functools.partial(_bq1_body, N, S, ks, r2s),
        out_type=[jax.ShapeDtypeStruct((B * S * k * 6,), jnp.float32)
                  for k in ks],
        mesh=plsc.VectorSubcoreMesh(**_SC_MESH),
        compiler_params=pltpu.CompilerParams(needs_layout_passes=False),
        scratch_types=[
            pltpu.VMEM((N,), jnp.float32), pltpu.VMEM((N,), jnp.float32),
            pltpu.VMEM((N,), jnp.float32), pltpu.VMEM((N,), jnp.float32),
            pltpu.VMEM((S // 4,), jnp.float32),
            pltpu.VMEM((S // 4,), jnp.float32),
            pltpu.VMEM((S // 4,), jnp.float32),
            pltpu.VMEM((S // 4,), jnp.float32),
            pltpu.VMEM((ks[0],), jnp.int32),
            pltpu.VMEM((ks[1],), jnp.int32),
            pltpu.VMEM((ks[2],), jnp.int32),
            pltpu.VMEM((ks[2] * 6,), jnp.float32),
        ],
    )
    outs = kern(xs.reshape(-1), ys.reshape(-1), zs.reshape(-1),
                cx.reshape(-1), cy.reshape(-1), cz.reshape(-1),
                pc.reshape(-1))
    return [o.reshape(B, S * k * 6) for o, k in zip(outs, ks)]


def _bq2_body(N, S, ks, r2s, xs_h, ys_h, zs_h, cx_h, cy_h, cz_h, pc_h,
              o1_h, o2_h, o3_h,
              xs_v, ys_v, zs_v, px_v, cxs_v, cys_v, czs_v, pcs_v,
              i1_v, i2_v, i3_v):
    k1, k2, k3 = ks
    r21, r22, r23 = r2s
    w = lax.axis_index("s") * 2 + lax.axis_index("c")
    b = w // 4
    q = w % 4
    S_sub = S // 4
    s0 = q * S_sub
    pltpu.sync_copy(xs_h.at[pl.ds(b * N, N)], xs_v)
    pltpu.sync_copy(ys_h.at[pl.ds(b * N, N)], ys_v)
    pltpu.sync_copy(zs_h.at[pl.ds(b * N, N)], zs_v)
    pltpu.sync_copy(cx_h.at[pl.ds(b * S + s0, S_sub)], cxs_v)
    pltpu.sync_copy(cy_h.at[pl.ds(b * S + s0, S_sub)], cys_v)
    pltpu.sync_copy(cz_h.at[pl.ds(b * S + s0, S_sub)], czs_v)
    pltpu.sync_copy(pc_h.at[pl.ds(b * S + s0, S_sub)], pcs_v)
    iota = lax.iota(jnp.int32, 16)

    def px_step(t, _):
        sl = pl.ds(t * 16, 16)
        xv = xs_v[sl]
        yv = ys_v[sl]
        zv = zs_v[sl]
        px_v[sl] = (xv * xv + yv * yv) + zv * zv
        return 0

    lax.fori_loop(0, N // 16, px_step, 0)

    def scal(ref, i):
        chv = ref[pl.ds((i // 16) * 16, 16)]
        return jnp.sum(jnp.where(iota == i % 16, chv,
                                 jnp.zeros((16,), chv.dtype)))

    def row_fn(rl, _):
        cx_s = scal(cxs_v, rl)
        cy_s = scal(cys_v, rl)
        cz_s = scal(czs_v, rl)
        pc_s = scal(pcs_v, rl)

        def chunk(t, cnts):
            c1, c2, c3 = cnts
            sl = pl.ds(t * 16, 16)
            xv = xs_v[sl]
            yv = ys_v[sl]
            zv = zs_v[sl]
            pxv = px_v[sl]
            dot = (cx_s * xv + cy_s * yv) + cz_s * zv
            d = (pc_s + pxv) - 2.0 * dot
            nvec = t * 16 + iota

            def one(r2, kk, buf, cnt):
                m = d <= r2
                ci = jnp.cumsum(m.astype(jnp.int32))
                pos = cnt + ci - 1
                wr = m & (pos < kk)
                plsc.store_scatter(buf, [jnp.maximum(pos, 0)], nvec,
                                   mask=wr)
                return cnt + jnp.sum(m.astype(jnp.int32))

            c1 = one(r21, k1, i1_v, c1)
            c2 = one(r22, k2, i2_v, c2)
            c3 = one(r23, k3, i3_v, c3)
            return (c1, c2, c3)

        zero = jnp.array(0, jnp.int32)
        c1, c2, c3 = lax.fori_loop(0, N // 16, chunk, (zero, zero, zero))
        sg = s0 + rl

        def emit(kk, cnt, buf, out_h):
            first = scal(buf, 0)

            def fill(ch, _):
                slot = ch * 16 + iota
                cur = buf[pl.ds(ch * 16, 16)]
                buf[pl.ds(ch * 16, 16)] = jnp.where(slot < cnt, cur, first)
                return 0

            lax.fori_loop(0, kk // 16, fill, 0)
            pltpu.sync_copy(buf, out_h.at[pl.ds((b * S + sg) * kk, kk)])

        emit(k1, c1, i1_v, o1_h)
        emit(k2, c2, i2_v, o2_h)
        emit(k3, c3, i3_v, o3_h)
        return 0

    lax.fori_loop(0, S_sub, row_fn, 0)


def _ball_query2(xs, ys, zs, cx, cy, cz, pc, ks, radii):
    B, N = xs.shape
    S = cx.shape[1]
    r2s = tuple(float(r) * float(r) for r in radii)
    kern = pl.kernel(
        functools.partial(_bq2_body, N, S, ks, r2s),
        out_type=[jax.ShapeDtypeStruct((B * S * k,), jnp.int32) for k in ks],
        mesh=plsc.VectorSubcoreMesh(**_SC_MESH),
        compiler_params=pltpu.CompilerParams(needs_layout_passes=False),
        scratch_types=[
            pltpu.VMEM((N,), jnp.float32), pltpu.VMEM((N,), jnp.float32),
            pltpu.VMEM((N,), jnp.float32), pltpu.VMEM((N,), jnp.float32),
            pltpu.VMEM((S // 4,), jnp.float32),
            pltpu.VMEM((S // 4,), jnp.float32),
            pltpu.VMEM((S // 4,), jnp.float32),
            pltpu.VMEM((S // 4,), jnp.float32),
            pltpu.VMEM((ks[0],), jnp.int32),
            pltpu.VMEM((ks[1],), jnp.int32),
            pltpu.VMEM((ks[2],), jnp.int32),
        ],
    )
    outs = kern(xs.reshape(-1), ys.reshape(-1), zs.reshape(-1),
                cx.reshape(-1), cy.reshape(-1), cz.reshape(-1),
                pc.reshape(-1))
    return [o.reshape(B, S * k) for o, k in zip(outs, ks)]


def _mlp1_body(K, st, g_ref, w1_ref, b1_ref, w2_ref, b2_ref, w3_ref, b3_ref,
               out_ref):
    g = g_ref[...].reshape(st * K, 6)
    h = jnp.maximum(jnp.dot(g, w1_ref[...]) + b1_ref[...], 0.0)
    h = jnp.maximum(jnp.dot(h, w2_ref[...]) + b2_ref[...], 0.0)
    h = jnp.maximum(jnp.dot(h, w3_ref[...]) + b3_ref[...], 0.0)
    c3 = h.shape[-1]
    out_ref[...] = jnp.max(h.reshape(st, K, c3), axis=1)[None]


def _mlp1(g, params, K, st):
    B = g.shape[0]
    S = g.shape[1] // (K * 6)
    (w1, b1), (w2, b2), (w3, b3) = params
    c3 = w3.shape[0]
    grid = (B, S // st)
    return pl.pallas_call(
        functools.partial(_mlp1_body, K, st),
        grid=grid,
        in_specs=[
            pl.BlockSpec((1, st * K, 6), lambda b, i: (b, i, 0)),
            pl.BlockSpec(w1.T.shape, lambda b, i: (0, 0)),
            pl.BlockSpec(b1.shape, lambda b, i: (0,)),
            pl.BlockSpec(w2.T.shape, lambda b, i: (0, 0)),
            pl.BlockSpec(b2.shape, lambda b, i: (0,)),
            pl.BlockSpec(w3.T.shape, lambda b, i: (0, 0)),
            pl.BlockSpec(b3.shape, lambda b, i: (0,)),
        ],
        out_specs=pl.BlockSpec((1, st, c3), lambda b, i: (b, i, 0)),
        out_shape=jax.ShapeDtypeStruct((B, S, c3), jnp.float32),
    )(g.reshape(B, S * K, 6), w1.T, b1, w2.T, b2, w3.T, b3)


def _pre1_body(f_ref, w_ref, b_ref, out_ref):
    out_ref[...] = (jnp.dot(f_ref[...][0], w_ref[...]) + b_ref[...])[None]


def _pre1(f1, w, b):
    B, S, C = f1.shape
    Co = w.shape[1]
    return pl.pallas_call(
        _pre1_body,
        grid=(B,),
        in_specs=[
            pl.BlockSpec((1, S, C), lambda b: (b, 0, 0)),
            pl.BlockSpec((C, Co), lambda b: (0, 0)),
            pl.BlockSpec((Co,), lambda b: (0,)),
        ],
        out_specs=pl.BlockSpec((1, S, Co), lambda b: (b, 0, 0)),
        out_shape=jax.ShapeDtypeStruct((B, S, Co), jnp.float32),
    )(f1, w, b)


def _mlp2_body(K, st, S1, idx_ref, pre_ref, xyz_ref, c_ref, w1x_ref,
               w2_ref, b2_ref, w3_ref, b3_ref, out_ref):
    rows = st * K
    idx = idx_ref[...].reshape(rows, 1)
    onehot = (idx == lax.broadcasted_iota(jnp.int32, (rows, S1), 1)
              ).astype(jnp.float32)
    gpre = jnp.dot(onehot, pre_ref[...][0])
    gxyz = jnp.dot(onehot, xyz_ref[...][0])
    cc = c_ref[...].reshape(st, 1, 3)
    dxyz = (gxyz.reshape(st, K, 3) - cc).reshape(rows, 3)
    h = jnp.maximum(gpre + jnp.dot(dxyz, w1x_ref[...]), 0.0)
    h = jnp.maximum(jnp.dot(h, w2_ref[...]) + b2_ref[...], 0.0)
    h = jnp.maximum(jnp.dot(h, w3_ref[...]) + b3_ref[...], 0.0)
    c3 = h.shape[-1]
    out_ref[...] = jnp.max(h.reshape(st, K, c3), axis=1)[None]


def _mlp2(idx, pre, xyz1, crows, params, K, st):
    B, S1, C1 = pre.shape
    S = idx.shape[1] // K
    (w1, _), (w2, b2), (w3, b3) = params
    w1x = w1[:, :3]
    c3 = w3.shape[0]
    grid = (B, S // st)
    return pl.pallas_call(
        functools.partial(_mlp2_body, K, st, S1),
        grid=grid,
        in_specs=[
            pl.BlockSpec((1, st * K, 1), lambda b, i: (b, i, 0)),
            pl.BlockSpec((1, S1, C1), lambda b, i: (b, 0, 0)),
            pl.BlockSpec((1, S1, 3), lambda b, i: (b, 0, 0)),
            pl.BlockSpec((1, st, 3), lambda b, i: (b, i, 0)),
            pl.BlockSpec((3, w1x.shape[0]), lambda b, i: (0, 0)),
            pl.BlockSpec(w2.T.shape, lambda b, i: (0, 0)),
            pl.BlockSpec(b2.shape, lambda b, i: (0,)),
            pl.BlockSpec(w3.T.shape, lambda b, i: (0, 0)),
            pl.BlockSpec(b3.shape, lambda b, i: (0,)),
        ],
        out_specs=pl.BlockSpec((1, st, c3), lambda b, i: (b, i, 0)),
        out_shape=jax.ShapeDtypeStruct((B, S, c3), jnp.float32),
    )(idx.reshape(B, S * K, 1), pre, xyz1, crows, w1x.T, w2.T, b2, w3.T, b3)


def _sa3_body(g_ref, w1_ref, b1_ref, w2_ref, b2_ref, w3_ref, b3_ref, out_ref):
    g = g_ref[...][0]
    h = jnp.maximum(jnp.dot(g, w1_ref[...]) + b1_ref[...], 0.0)
    h = jnp.maximum(jnp.dot(h, w2_ref[...]) + b2_ref[...], 0.0)
    h = jnp.maximum(jnp.dot(h, w3_ref[...]) + b3_ref[...], 0.0)
    out_ref[...] = jnp.max(h, axis=0).reshape(1, 1, -1)


def _sa3(g, params):
    B, S, C = g.shape
    (w1, b1), (w2, b2), (w3, b3) = params
    c3 = w3.shape[0]
    return pl.pallas_call(
        _sa3_body,
        grid=(B,),
        in_specs=[
            pl.BlockSpec((1, S, C), lambda b: (b, 0, 0)),
            pl.BlockSpec(w1.T.shape, lambda b: (0, 0)),
            pl.BlockSpec(b1.shape, lambda b: (0,)),
            pl.BlockSpec(w2.T.shape, lambda b: (0, 0)),
            pl.BlockSpec(b2.shape, lambda b: (0,)),
            pl.BlockSpec(w3.T.shape, lambda b: (0, 0)),
            pl.BlockSpec(b3.shape, lambda b: (0,)),
        ],
        out_specs=pl.BlockSpec((1, 1, c3), lambda b: (b, 0, 0)),
        out_shape=jax.ShapeDtypeStruct((B, 1, c3), jnp.float32),
    )(g, w1.T, b1, w2.T, b2, w3.T, b3).reshape(B, c3)


def _head_body(f_ref, w1_ref, b1_ref, w2_ref, b2_ref, w3_ref, b3_ref, out_ref):
    f = f_ref[...]
    f = jnp.maximum(jnp.dot(f, w1_ref[...]) + b1_ref[...], 0.0)
    f = jnp.maximum(jnp.dot(f, w2_ref[...]) + b2_ref[...], 0.0)
    f = jnp.dot(f, w3_ref[...]) + b3_ref[...]
    out_ref[...] = jax.nn.log_softmax(f, axis=-1)


def _head(feat, params):
    (w1, b1), (w2, b2), (w3, b3) = params
    return pl.pallas_call(
        _head_body,
        out_shape=jax.ShapeDtypeStruct((feat.shape[0], w3.shape[0]),
                                       jnp.float32),
    )(feat, w1.T, b1, w2.T, b2, w3.T, b3)


def kernel(x, sa1_params, sa2_params, sa3_params, head_params):
    B = x.shape[0]
    xs, ys, zs = x[:, 0, :], x[:, 1, :], x[:, 2, :]

    k1 = (16, 32, 128)
    r1 = (0.1, 0.2, 0.4)
    cx1, cy1, cz1, pc1 = _fps(xs, ys, zs, 512)
    g1a, g1b, g1c = _ball_group1(xs, ys, zs, cx1, cy1, cz1, pc1, k1, r1)
    f1a = _mlp1(g1a, sa1_params[0], 16, 64)
    f1b = _mlp1(g1b, sa1_params[1], 32, 32)
    f1c = _mlp1(g1c, sa1_params[2], 128, 16)
    f1 = jnp.concatenate([f1a, f1b, f1c], axis=-1)

    k2 = (32, 64, 128)
    r2 = (0.2, 0.4, 0.8)
    cx2, cy2, cz2, pc2 = _fps(cx1, cy1, cz1, 128)
    i2a, i2b, i2c = _ball_query2(cx1, cy1, cz1, cx2, cy2, cz2, pc2, k2, r2)
    w1f = jnp.concatenate([p[0][0][:, 3:] for p in sa2_params], axis=0)
    b1f = jnp.concatenate([p[0][1] for p in sa2_params], axis=0)
    pre1 = _pre1(f1, w1f.T, b1f)
    xyz1 = jnp.stack([cx1, cy1, cz1], axis=-1)
    xyz2 = jnp.stack([cx2, cy2, cz2], axis=-1)
    offs = [0, 64, 192]
    widths = [64, 128, 128]
    f2s = []
    for i, (idx, K, st) in enumerate(zip([i2a, i2b, i2c], k2, [16, 8, 8])):
        pre_i = lax.slice_in_dim(pre1, offs[i], offs[i] + widths[i], axis=2)
        f2s.append(_mlp2(idx, pre_i, xyz1, xyz2, sa2_params[i], K, st))
    f2 = jnp.concatenate(f2s, axis=-1)

    g3 = jnp.concatenate([xyz2, f2], axis=-1)
    f3 = _sa3(g3, sa3_params)
    return _head(f3, head_params)


# speedup vs baseline: 15.3456x; 1.0779x over previous
import functools

import jax
import jax.numpy as jnp
from jax import lax
from jax.experimental import pallas as pl
from jax.experimental.pallas import tpu as pltpu
from jax.experimental.pallas import tpu_sc as plsc


def _fps_body(npoint, xs_ref, ys_ref, zs_ref, ox_ref, oy_ref, oz_ref,
              opc_ref, dist_ref):
    B, N = xs_ref.shape
    xs = xs_ref[...]
    ys = ys_ref[...]
    zs = zs_ref[...]
    lanes = lax.broadcasted_iota(jnp.int32, (B, N), 1)
    out_lanes = lax.broadcasted_iota(jnp.int32, (B, npoint), 1)
    dist_ref[...] = jnp.full((B, N), 1e10, jnp.float32)

    def step(i, far):
        onehot = lanes == far
        cx = jnp.sum(jnp.where(onehot, xs, 0.0), axis=1, keepdims=True)
        cy = jnp.sum(jnp.where(onehot, ys, 0.0), axis=1, keepdims=True)
        cz = jnp.sum(jnp.where(onehot, zs, 0.0), axis=1, keepdims=True)
        dx = xs - cx
        dy = ys - cy
        dz = zs - cz
        d = (dx * dx + dy * dy) + dz * dz
        dist = jnp.minimum(dist_ref[...], d)
        dist_ref[...] = dist
        m = jnp.max(dist, axis=1, keepdims=True)
        far_new = jnp.min(jnp.where(dist == m, lanes, N), axis=1,
                          keepdims=True).astype(jnp.int32)
        sel = out_lanes == i
        ox_ref[...] = jnp.where(sel, cx, ox_ref[...])
        oy_ref[...] = jnp.where(sel, cy, oy_ref[...])
        oz_ref[...] = jnp.where(sel, cz, oz_ref[...])
        opc_ref[...] = jnp.where(sel, (cx * cx + cy * cy) + cz * cz,
                                 opc_ref[...])
        return far_new

    lax.fori_loop(0, npoint, step, jnp.zeros((B, 1), jnp.int32))


def _fps(xs, ys, zs, npoint):
    B, N = xs.shape
    out = jax.ShapeDtypeStruct((B, npoint), jnp.float32)
    return pl.pallas_call(
        functools.partial(_fps_body, npoint),
        out_shape=[out, out, out, out],
        scratch_shapes=[pltpu.VMEM((B, N), jnp.float32)],
    )(xs, ys, zs)


_SC_MESH = dict(core_axis_name="c", subcore_axis_name="s")


def _bq1_body(N, S, ks, r2s, xs_h, ys_h, zs_h, cx_h, cy_h, cz_h, pc_h,
              g1_h, g2_h, g3_h,
              xs_v, ys_v, zs_v, px_v, cxs_v, cys_v, czs_v, pcs_v,
              i1_v, i2_v, i3_v, row_v):
    k1, k2, k3 = ks
    r21, r22, r23 = r2s
    w = lax.axis_index("s") * 2 + lax.axis_index("c")
    b = w // 4
    q = w % 4
    S_sub = S // 4
    s0 = q * S_sub
    pltpu.sync_copy(xs_h.at[pl.ds(b * N, N)], xs_v)
    pltpu.sync_copy(ys_h.at[pl.ds(b * N, N)], ys_v)
    pltpu.sync_copy(zs_h.at[pl.ds(b * N, N)], zs_v)
    pltpu.sync_copy(cx_h.at[pl.ds(b * S + s0, S_sub)], cxs_v)
    pltpu.sync_copy(cy_h.at[pl.ds(b * S + s0, S_sub)], cys_v)
    pltpu.sync_copy(cz_h.at[pl.ds(b * S + s0, S_sub)], czs_v)
    pltpu.sync_copy(pc_h.at[pl.ds(b * S + s0, S_sub)], pcs_v)
    iota = lax.iota(jnp.int32, 16)

    def px_step(t, _):
        sl = pl.ds(t * 16, 16)
        xv = xs_v[sl]
        yv = ys_v[sl]
        zv = zs_v[sl]
        px_v[sl] = (xv * xv + yv * yv) + zv * zv
        return 0

    lax.fori_loop(0, N // 16, px_step, 0)

    def scal(ref, i):
        chv = ref[pl.ds((i // 16) * 16, 16)]
        return jnp.sum(jnp.where(iota == i % 16, chv,
                                 jnp.zeros((16,), chv.dtype)))

    def row_fn(rl, _):
        cx_s = scal(cxs_v, rl)
        cy_s = scal(cys_v, rl)
        cz_s = scal(czs_v, rl)
        pc_s = scal(pcs_v, rl)

        def chunk(t, cnts):
            c1, c2, c3 = cnts
            sl = pl.ds(t * 16, 16)
            xv = xs_v[sl]
            yv = ys_v[sl]
            zv = zs_v[sl]
            pxv = px_v[sl]
            dot = (cx_s * xv + cy_s * yv) + cz_s * zv
            d = (pc_s + pxv) - 2.0 * dot
            nvec = t * 16 + iota

            def one(r2, kk, buf, cnt):
                m = d <= r2
                ci = jnp.cumsum(m.astype(jnp.int32))
                pos = cnt + ci - 1
                wr = m & (pos < kk)
                plsc.store_scatter(buf, [jnp.maximum(pos, 0)], nvec, mask=wr)
                return cnt + jnp.sum(m.astype(jnp.int32))

            c1 = one(r21, k1, i1_v, c1)
            c2 = one(r22, k2, i2_v, c2)
            c3 = one(r23, k3, i3_v, c3)
            return (c1, c2, c3)

        zero = jnp.array(0, jnp.int32)
        c1, c2, c3 = lax.fori_loop(0, N // 16, chunk, (zero, zero, zero))
        sg = s0 + rl

        def emit(kk, cnt, buf, out_h):
            first = scal(buf, 0)

            def fill(ch, _):
                slot = ch * 16 + iota
                cur = buf[pl.ds(ch * 16, 16)]
                vals = jnp.where(slot < cnt, cur, first)
                gx = plsc.load_gather(xs_v, [vals])
                gy = plsc.load_gather(ys_v, [vals])
                gz = plsc.load_gather(zs_v, [vals])
                base = slot * 6
                plsc.store_scatter(row_v, [base], gx - cx_s)
                plsc.store_scatter(row_v, [base + 1], gy - cy_s)
                plsc.store_scatter(row_v, [base + 2], gz - cz_s)
                plsc.store_scatter(row_v, [base + 3], gx)
                plsc.store_scatter(row_v, [base + 4], gy)
                plsc.store_scatter(row_v, [base + 5], gz)
                return 0

            lax.fori_loop(0, kk // 16, fill, 0)
            pltpu.sync_copy(row_v.at[pl.ds(0, kk * 6)],
                            out_h.at[pl.ds((b * S + sg) * kk * 6, kk * 6)])

        emit(k1, c1, i1_v, g1_h)
        emit(k2, c2, i2_v, g2_h)
        emit(k3, c3, i3_v, g3_h)
        return 0

    lax.fori_loop(0, S_sub, row_fn, 0)


def _ball_group1(xs, ys, zs, cx, cy, cz, pc, ks, radii):
    B, N = xs.shape
    S = cx.shape[1]
    r2s = tuple(float(r) * float(r) for r in radii)
    kern = pl.kernel(
        functools.partial(_bq1_body, N, S, ks, r2s),
        out_type=[jax.ShapeDtypeStruct((B * S * k * 6,), jnp.float32)
                  for k in ks],
        mesh=plsc.VectorSubcoreMesh(**_SC_MESH),
        compiler_params=pltpu.CompilerParams(needs_layout_passes=False),
        scratch_types=[
            pltpu.VMEM((N,), jnp.float32), pltpu.VMEM((N,), jnp.float32),
            pltpu.VMEM((N,), jnp.float32), pltpu.VMEM((N,), jnp.float32),
            pltpu.VMEM((S // 4,), jnp.float32),
            pltpu.VMEM((S // 4,), jnp.float32),
            pltpu.VMEM((S // 4,), jnp.float32),
            pltpu.VMEM((S // 4,), jnp.float32),
            pltpu.VMEM((ks[0],), jnp.int32),
            pltpu.VMEM((ks[1],), jnp.int32),
            pltpu.VMEM((ks[2],), jnp.int32),
            pltpu.VMEM((ks[2] * 6,), jnp.float32),
        ],
    )
    outs = kern(xs.reshape(-1), ys.reshape(-1), zs.reshape(-1),
                cx.reshape(-1), cy.reshape(-1), cz.reshape(-1),
                pc.reshape(-1))
    return [o.reshape(B, S * k * 6) for o, k in zip(outs, ks)]


def _bq2_body(N, S, ks, r2s, xs_h, ys_h, zs_h, cx_h, cy_h, cz_h, pc_h,
              o1_h, o2_h, o3_h,
              xs_v, ys_v, zs_v, px_v, cxs_v, cys_v, czs_v, pcs_v,
              i1_v, i2_v, i3_v):
    k1, k2, k3 = ks
    r21, r22, r23 = r2s
    w = lax.axis_index("s") * 2 + lax.axis_index("c")
    b = w // 4
    q = w % 4
    S_sub = S // 4
    s0 = q * S_sub
    pltpu.sync_copy(xs_h.at[pl.ds(b * N, N)], xs_v)
    pltpu.sync_copy(ys_h.at[pl.ds(b * N, N)], ys_v)
    pltpu.sync_copy(zs_h.at[pl.ds(b * N, N)], zs_v)
    pltpu.sync_copy(cx_h.at[pl.ds(b * S + s0, S_sub)], cxs_v)
    pltpu.sync_copy(cy_h.at[pl.ds(b * S + s0, S_sub)], cys_v)
    pltpu.sync_copy(cz_h.at[pl.ds(b * S + s0, S_sub)], czs_v)
    pltpu.sync_copy(pc_h.at[pl.ds(b * S + s0, S_sub)], pcs_v)
    iota = lax.iota(jnp.int32, 16)

    def px_step(t, _):
        sl = pl.ds(t * 16, 16)
        xv = xs_v[sl]
        yv = ys_v[sl]
        zv = zs_v[sl]
        px_v[sl] = (xv * xv + yv * yv) + zv * zv
        return 0

    lax.fori_loop(0, N // 16, px_step, 0)

    def scal(ref, i):
        chv = ref[pl.ds((i // 16) * 16, 16)]
        return jnp.sum(jnp.where(iota == i % 16, chv,
                                 jnp.zeros((16,), chv.dtype)))

    def row_fn(rl, _):
        cx_s = scal(cxs_v, rl)
        cy_s = scal(cys_v, rl)
        cz_s = scal(czs_v, rl)
        pc_s = scal(pcs_v, rl)

        def chunk(t, cnts):
            c1, c2, c3 = cnts
            sl = pl.ds(t * 16, 16)
            xv = xs_v[sl]
            yv = ys_v[sl]
            zv = zs_v[sl]
            pxv = px_v[sl]
            dot = (cx_s * xv + cy_s * yv) + cz_s * zv
            d = (pc_s + pxv) - 2.0 * dot
            nvec = t * 16 + iota

            def one(r2, kk, buf, cnt):
                m = d <= r2
                ci = jnp.cumsum(m.astype(jnp.int32))
                pos = cnt + ci - 1
                wr = m & (pos < kk)
                plsc.store_scatter(buf, [jnp.maximum(pos, 0)], nvec,
                                   mask=wr)
                return cnt + jnp.sum(m.astype(jnp.int32))

            c1 = one(r21, k1, i1_v, c1)
            c2 = one(r22, k2, i2_v, c2)
            c3 = one(r23, k3, i3_v, c3)
            return (c1, c2, c3)

        zero = jnp.array(0, jnp.int32)
        c1, c2, c3 = lax.fori_loop(0, N // 16, chunk, (zero, zero, zero))
        sg = s0 + rl

        def emit(kk, cnt, buf, out_h):
            first = scal(buf, 0)

            def fill(ch, _):
                slot = ch * 16 + iota
                cur = buf[pl.ds(ch * 16, 16)]
                buf[pl.ds(ch * 16, 16)] = jnp.where(slot < cnt, cur, first)
                return 0

            lax.fori_loop(0, kk // 16, fill, 0)
            pltpu.sync_copy(buf, out_h.at[pl.ds((b * S + sg) * kk, kk)])

        emit(k1, c1, i1_v, o1_h)
        emit(k2, c2, i2_v, o2_h)
        emit(k3, c3, i3_v, o3_h)
        return 0

    lax.fori_loop(0, S_sub, row_fn, 0)


def _ball_query2(xs, ys, zs, cx, cy, cz, pc, ks, radii):
    B, N = xs.shape
    S = cx.shape[1]
    r2s = tuple(float(r) * float(r) for r in radii)
    kern = pl.kernel(
        functools.partial(_bq2_body, N, S, ks, r2s),
        out_type=[jax.ShapeDtypeStruct((B * S * k,), jnp.int32) for k in ks],
        mesh=plsc.VectorSubcoreMesh(**_SC_MESH),
        compiler_params=pltpu.CompilerParams(needs_layout_passes=False),
        scratch_types=[
            pltpu.VMEM((N,), jnp.float32), pltpu.VMEM((N,), jnp.float32),
            pltpu.VMEM((N,), jnp.float32), pltpu.VMEM((N,), jnp.float32),
            pltpu.VMEM((S // 4,), jnp.float32),
            pltpu.VMEM((S // 4,), jnp.float32),
            pltpu.VMEM((S // 4,), jnp.float32),
            pltpu.VMEM((S // 4,), jnp.float32),
            pltpu.VMEM((ks[0],), jnp.int32),
            pltpu.VMEM((ks[1],), jnp.int32),
            pltpu.VMEM((ks[2],), jnp.int32),
        ],
    )
    outs = kern(xs.reshape(-1), ys.reshape(-1), zs.reshape(-1),
                cx.reshape(-1), cy.reshape(-1), cz.reshape(-1),
                pc.reshape(-1))
    return [o.reshape(B, S * k) for o, k in zip(outs, ks)]


def _mlp1_body(K, sp, g_ref, w1_ref, b1_ref, w2_ref, b2_ref, w3_ref, b3_ref,
               out_ref):
    g = g_ref[...].reshape(sp * K, 6)
    h = jnp.maximum(jnp.dot(g, w1_ref[...]) + b1_ref[...], 0.0)
    h = jnp.maximum(jnp.dot(h, w2_ref[...]) + b2_ref[...], 0.0)
    h = jnp.maximum(jnp.dot(h, w3_ref[...]) + b3_ref[...], 0.0)
    c3 = h.shape[-1]
    out_ref[...] = jnp.max(h.reshape(sp, K, c3), axis=1)[None]


def _mlp1(g, params, K, sp):
    B = g.shape[0]
    S = g.shape[1] // (K * 6)
    (w1, b1), (w2, b2), (w3, b3) = params
    c3 = w3.shape[0]
    grid = (B, S // sp)
    gv = g.reshape(B, S * K, 6)
    return pl.pallas_call(
        functools.partial(_mlp1_body, K, sp),
        grid=grid,
        in_specs=[
            pl.BlockSpec((1, sp * K, 6), lambda b, i: (b, i, 0)),
            pl.BlockSpec(w1.T.shape, lambda b, i: (0, 0)),
            pl.BlockSpec(b1.shape, lambda b, i: (0,)),
            pl.BlockSpec(w2.T.shape, lambda b, i: (0, 0)),
            pl.BlockSpec(b2.shape, lambda b, i: (0,)),
            pl.BlockSpec(w3.T.shape, lambda b, i: (0, 0)),
            pl.BlockSpec(b3.shape, lambda b, i: (0,)),
        ],
        out_specs=pl.BlockSpec((1, sp, c3), lambda b, i: (b, i, 0)),
        out_shape=jax.ShapeDtypeStruct((B, S, c3), jnp.float32),
    )(gv, w1.T, b1, w2.T, b2, w3.T, b3)


def _pre1_body(f_ref, w_ref, b_ref, out_ref):
    out_ref[...] = (jnp.dot(f_ref[...][0], w_ref[...]) + b_ref[...])[None]


def _pre1(f1, w, b):
    B, S, C = f1.shape
    Co = w.shape[1]
    return pl.pallas_call(
        _pre1_body,
        grid=(B,),
        in_specs=[
            pl.BlockSpec((1, S, C), lambda b: (b, 0, 0)),
            pl.BlockSpec((C, Co), lambda b: (0, 0)),
            pl.BlockSpec((Co,), lambda b: (0,)),
        ],
        out_specs=pl.BlockSpec((1, S, Co), lambda b: (b, 0, 0)),
        out_shape=jax.ShapeDtypeStruct((B, S, Co), jnp.float32),
    )(f1, w, b)


def _mlp2_body(K, sp, S1, idx_ref, pre_ref, xyz_ref, c_ref, w1x_ref,
               w2_ref, b2_ref, w3_ref, b3_ref, out_ref):
    rows = sp * K
    idxr = idx_ref[...].reshape(1, rows)
    gt = (lax.broadcasted_iota(jnp.int32, (S1, rows), 0) == idxr
          ).astype(jnp.float32)
    dn = (((0,), (0,)), ((), ()))
    gpre = lax.dot_general(gt, pre_ref[...][0], dimension_numbers=dn)
    gxyz = lax.dot_general(gt, xyz_ref[...][0], dimension_numbers=dn)
    cc = c_ref[...].reshape(sp, 1, 3)
    dxyz = (gxyz.reshape(sp, K, 3) - cc).reshape(rows, 3)
    h = jnp.maximum(gpre + jnp.dot(dxyz, w1x_ref[...]), 0.0)
    h = jnp.maximum(jnp.dot(h, w2_ref[...]) + b2_ref[...], 0.0)
    h = jnp.maximum(jnp.dot(h, w3_ref[...]) + b3_ref[...], 0.0)
    c3 = h.shape[-1]
    out_ref[...] = jnp.max(h.reshape(sp, K, c3), axis=1)[None]


def _mlp2(idx, pre, xyz1, crows, params, K, sp):
    B, S1, C1 = pre.shape
    S = idx.shape[1] // K
    (w1, _), (w2, b2), (w3, b3) = params
    w1x = w1[:, :3]
    c3 = w3.shape[0]
    grid = (B, S // sp)
    iv = idx.reshape(B, S // sp, 8, sp * K // 8)
    return pl.pallas_call(
        functools.partial(_mlp2_body, K, sp, S1),
        grid=grid,
        in_specs=[
            pl.BlockSpec((1, 1, 8, sp * K // 8), lambda b, i: (b, i, 0, 0)),
            pl.BlockSpec((1, S1, C1), lambda b, i: (b, 0, 0)),
            pl.BlockSpec((1, S1, 3), lambda b, i: (b, 0, 0)),
            pl.BlockSpec((1, sp, 3), lambda b, i: (b, i, 0)),
            pl.BlockSpec((3, w1x.shape[0]), lambda b, i: (0, 0)),
            pl.BlockSpec(w2.T.shape, lambda b, i: (0, 0)),
            pl.BlockSpec(b2.shape, lambda b, i: (0,)),
            pl.BlockSpec(w3.T.shape, lambda b, i: (0, 0)),
            pl.BlockSpec(b3.shape, lambda b, i: (0,)),
        ],
        out_specs=pl.BlockSpec((1, sp, c3), lambda b, i: (b, i, 0)),
        out_shape=jax.ShapeDtypeStruct((B, S, c3), jnp.float32),
    )(iv, pre, xyz1, crows, w1x.T, w2.T, b2, w3.T, b3)


def _sa3_body(g_ref, w1_ref, b1_ref, w2_ref, b2_ref, w3_ref, b3_ref, out_ref):
    g = g_ref[...][0]
    h = jnp.maximum(jnp.dot(g, w1_ref[...]) + b1_ref[...], 0.0)
    h = jnp.maximum(jnp.dot(h, w2_ref[...]) + b2_ref[...], 0.0)
    h = jnp.maximum(jnp.dot(h, w3_ref[...]) + b3_ref[...], 0.0)
    out_ref[...] = jnp.max(h, axis=0).reshape(1, 1, -1)


def _sa3(g, params):
    B, S, C = g.shape
    (w1, b1), (w2, b2), (w3, b3) = params
    c3 = w3.shape[0]
    return pl.pallas_call(
        _sa3_body,
        grid=(B,),
        in_specs=[
            pl.BlockSpec((1, S, C), lambda b: (b, 0, 0)),
            pl.BlockSpec(w1.T.shape, lambda b: (0, 0)),
            pl.BlockSpec(b1.shape, lambda b: (0,)),
            pl.BlockSpec(w2.T.shape, lambda b: (0, 0)),
            pl.BlockSpec(b2.shape, lambda b: (0,)),
            pl.BlockSpec(w3.T.shape, lambda b: (0, 0)),
            pl.BlockSpec(b3.shape, lambda b: (0,)),
        ],
        out_specs=pl.BlockSpec((1, 1, c3), lambda b: (b, 0, 0)),
        out_shape=jax.ShapeDtypeStruct((B, 1, c3), jnp.float32),
    )(g, w1.T, b1, w2.T, b2, w3.T, b3).reshape(B, c3)


def _head_body(f_ref, w1_ref, b1_ref, w2_ref, b2_ref, w3_ref, b3_ref, out_ref):
    f = f_ref[...]
    f = jnp.maximum(jnp.dot(f, w1_ref[...]) + b1_ref[...], 0.0)
    f = jnp.maximum(jnp.dot(f, w2_ref[...]) + b2_ref[...], 0.0)
    f = jnp.dot(f, w3_ref[...]) + b3_ref[...]
    out_ref[...] = jax.nn.log_softmax(f, axis=-1)


def _head(feat, params):
    (w1, b1), (w2, b2), (w3, b3) = params
    return pl.pallas_call(
        _head_body,
        out_shape=jax.ShapeDtypeStruct((feat.shape[0], w3.shape[0]),
                                       jnp.float32),
    )(feat, w1.T, b1, w2.T, b2, w3.T, b3)


def kernel(x, sa1_params, sa2_params, sa3_params, head_params):
    B = x.shape[0]
    xs, ys, zs = x[:, 0, :], x[:, 1, :], x[:, 2, :]

    k1 = (16, 32, 128)
    r1 = (0.1, 0.2, 0.4)
    cx1, cy1, cz1, pc1 = _fps(xs, ys, zs, 512)
    g1a, g1b, g1c = _ball_group1(xs, ys, zs, cx1, cy1, cz1, pc1, k1, r1)
    f1a = _mlp1(g1a, sa1_params[0], 16, 64)
    f1b = _mlp1(g1b, sa1_params[1], 32, 32)
    f1c = _mlp1(g1c, sa1_params[2], 128, 16)
    f1 = jnp.concatenate([f1a, f1b, f1c], axis=-1)

    k2 = (32, 64, 128)
    r2 = (0.2, 0.4, 0.8)
    cx2, cy2, cz2, pc2 = _fps(cx1, cy1, cz1, 128)
    i2a, i2b, i2c = _ball_query2(cx1, cy1, cz1, cx2, cy2, cz2, pc2, k2, r2)
    w1f = jnp.concatenate([p[0][0][:, 3:] for p in sa2_params], axis=0)
    b1f = jnp.concatenate([p[0][1] for p in sa2_params], axis=0)
    pre1 = _pre1(f1, w1f.T, b1f)
    xyz1 = jnp.stack([cx1, cy1, cz1], axis=-1)
    xyz2 = jnp.stack([cx2, cy2, cz2], axis=-1)
    offs = [0, 64, 192]
    widths = [64, 128, 128]
    f2s = []
    for i, (idx, K, st) in enumerate(zip([i2a, i2b, i2c], k2, [32, 16, 8])):
        pre_i = lax.slice_in_dim(pre1, offs[i], offs[i] + widths[i], axis=2)
        f2s.append(_mlp2(idx, pre_i, xyz1, xyz2, sa2_params[i], K, st))
    f2 = jnp.concatenate(f2s, axis=-1)

    g3 = jnp.concatenate([xyz2, f2], axis=-1)
    f3 = _sa3(g3, sa3_params)
    return _head(f3, head_params)


# speedup vs baseline: 19.0205x; 1.2395x over previous
import functools

import jax
import jax.numpy as jnp
from jax import lax
from jax.experimental import pallas as pl
from jax.experimental.pallas import tpu as pltpu
from jax.experimental.pallas import tpu_sc as plsc


def _fps_body(npoint, xs_ref, ys_ref, zs_ref, ox_ref, oy_ref, oz_ref,
              opc_ref, dist_ref):
    B, N = xs_ref.shape
    xs = xs_ref[...]
    ys = ys_ref[...]
    zs = zs_ref[...]
    lanes = lax.broadcasted_iota(jnp.int32, (B, N), 1)
    out_lanes = lax.broadcasted_iota(jnp.int32, (B, npoint), 1)
    dist_ref[...] = jnp.full((B, N), 1e10, jnp.float32)

    def step(i, far):
        onehot = lanes == far
        cx = jnp.sum(jnp.where(onehot, xs, 0.0), axis=1, keepdims=True)
        cy = jnp.sum(jnp.where(onehot, ys, 0.0), axis=1, keepdims=True)
        cz = jnp.sum(jnp.where(onehot, zs, 0.0), axis=1, keepdims=True)
        dx = xs - cx
        dy = ys - cy
        dz = zs - cz
        d = (dx * dx + dy * dy) + dz * dz
        dist = jnp.minimum(dist_ref[...], d)
        dist_ref[...] = dist
        m = jnp.max(dist, axis=1, keepdims=True)
        far_new = jnp.min(jnp.where(dist == m, lanes, N), axis=1,
                          keepdims=True).astype(jnp.int32)
        sel = out_lanes == i
        ox_ref[...] = jnp.where(sel, cx, ox_ref[...])
        oy_ref[...] = jnp.where(sel, cy, oy_ref[...])
        oz_ref[...] = jnp.where(sel, cz, oz_ref[...])
        opc_ref[...] = jnp.where(sel, (cx * cx + cy * cy) + cz * cz,
                                 opc_ref[...])
        return far_new

    lax.fori_loop(0, npoint, step, jnp.zeros((B, 1), jnp.int32))


def _fps(xs, ys, zs, npoint):
    B, N = xs.shape
    out = jax.ShapeDtypeStruct((B, npoint), jnp.float32)
    return pl.pallas_call(
        functools.partial(_fps_body, npoint),
        out_shape=[out, out, out, out],
        scratch_shapes=[pltpu.VMEM((B, N), jnp.float32)],
    )(xs, ys, zs)


_SC_MESH = dict(core_axis_name="c", subcore_axis_name="s")


def _bq1_body(N, S, ks, r2s, sps, xs_h, ys_h, zs_h, cx_h, cy_h, cz_h, pc_h,
              g1_h, g2_h, g3_h,
              xs_v, ys_v, zs_v, px_v, cxs_v, cys_v, czs_v, pcs_v,
              i1_v, i2_v, i3_v, t1_v, t2_v, t3_v):
    k1, k2, k3 = ks
    sp1, sp2, sp3 = sps
    r21, r22, r23 = r2s
    w = lax.axis_index("s") * 2 + lax.axis_index("c")
    b = w // 4
    q = w % 4
    S_sub = S // 4
    s0 = q * S_sub
    pltpu.sync_copy(xs_h.at[pl.ds(b * N, N)], xs_v)
    pltpu.sync_copy(ys_h.at[pl.ds(b * N, N)], ys_v)
    pltpu.sync_copy(zs_h.at[pl.ds(b * N, N)], zs_v)
    pltpu.sync_copy(cx_h.at[pl.ds(b * S + s0, S_sub)], cxs_v)
    pltpu.sync_copy(cy_h.at[pl.ds(b * S + s0, S_sub)], cys_v)
    pltpu.sync_copy(cz_h.at[pl.ds(b * S + s0, S_sub)], czs_v)
    pltpu.sync_copy(pc_h.at[pl.ds(b * S + s0, S_sub)], pcs_v)
    iota = lax.iota(jnp.int32, 16)

    def px_step(t, _):
        sl = pl.ds(t * 16, 16)
        xv = xs_v[sl]
        yv = ys_v[sl]
        zv = zs_v[sl]
        px_v[sl] = (xv * xv + yv * yv) + zv * zv
        return 0

    lax.fori_loop(0, N // 16, px_step, 0)

    def scal(ref, i):
        chv = ref[pl.ds((i // 16) * 16, 16)]
        return jnp.sum(jnp.where(iota == i % 16, chv,
                                 jnp.zeros((16,), chv.dtype)))

    def row_fn(rl, _):
        cx_s = scal(cxs_v, rl)
        cy_s = scal(cys_v, rl)
        cz_s = scal(czs_v, rl)
        pc_s = scal(pcs_v, rl)

        def chunk(t, cnts):
            c1, c2, c3 = cnts
            sl = pl.ds(t * 16, 16)
            xv = xs_v[sl]
            yv = ys_v[sl]
            zv = zs_v[sl]
            pxv = px_v[sl]
            dot = (cx_s * xv + cy_s * yv) + cz_s * zv
            d = (pc_s + pxv) - 2.0 * dot
            nvec = t * 16 + iota

            def one(r2, kk, buf, cnt):
                m = d <= r2
                ci = jnp.cumsum(m.astype(jnp.int32))
                pos = cnt + ci - 1
                wr = m & (pos < kk)
                plsc.store_scatter(buf, [jnp.maximum(pos, 0)], nvec, mask=wr)
                return cnt + jnp.sum(m.astype(jnp.int32))

            c1 = one(r21, k1, i1_v, c1)
            c2 = one(r22, k2, i2_v, c2)
            c3 = one(r23, k3, i3_v, c3)
            return (c1, c2, c3)

        zero = jnp.array(0, jnp.int32)
        c1, c2, c3 = lax.fori_loop(0, N // 16, chunk, (zero, zero, zero))
        sg = s0 + rl

        def emit(kk, sp, cnt, buf, tile_v, out_h):
            first = scal(buf, 0)
            spk = sp * kk
            rmod = sg % sp

            def fill(ch, _):
                slot = ch * 16 + iota
                cur = buf[pl.ds(ch * 16, 16)]
                vals = jnp.where(slot < cnt, cur, first)
                gx = plsc.load_gather(xs_v, [vals])
                gy = plsc.load_gather(ys_v, [vals])
                gz = plsc.load_gather(zs_v, [vals])
                base = rmod * kk + slot
                plsc.store_scatter(tile_v, [base], gx - cx_s)
                plsc.store_scatter(tile_v, [base + spk], gy - cy_s)
                plsc.store_scatter(tile_v, [base + 2 * spk], gz - cz_s)
                plsc.store_scatter(tile_v, [base + 3 * spk], gx)
                plsc.store_scatter(tile_v, [base + 4 * spk], gy)
                plsc.store_scatter(tile_v, [base + 5 * spk], gz)
                return 0

            lax.fori_loop(0, kk // 16, fill, 0)

            @pl.when(rmod == sp - 1)
            def _():
                tile = sg // sp
                pltpu.sync_copy(
                    tile_v,
                    out_h.at[pl.ds((b * (S // sp) + tile) * 6 * spk, 6 * spk)])

        emit(k1, sp1, c1, i1_v, t1_v, g1_h)
        emit(k2, sp2, c2, i2_v, t2_v, g2_h)
        emit(k3, sp3, c3, i3_v, t3_v, g3_h)
        return 0

    lax.fori_loop(0, S_sub, row_fn, 0)


def _ball_group1(xs, ys, zs, cx, cy, cz, pc, ks, radii, sps):
    B, N = xs.shape
    S = cx.shape[1]
    r2s = tuple(float(r) * float(r) for r in radii)
    kern = pl.kernel(
        functools.partial(_bq1_body, N, S, ks, r2s, sps),
        out_type=[jax.ShapeDtypeStruct((B * S * k * 6,), jnp.float32)
                  for k in ks],
        mesh=plsc.VectorSubcoreMesh(**_SC_MESH),
        compiler_params=pltpu.CompilerParams(needs_layout_passes=False),
        scratch_types=[
            pltpu.VMEM((N,), jnp.float32), pltpu.VMEM((N,), jnp.float32),
            pltpu.VMEM((N,), jnp.float32), pltpu.VMEM((N,), jnp.float32),
            pltpu.VMEM((S // 4,), jnp.float32),
            pltpu.VMEM((S // 4,), jnp.float32),
            pltpu.VMEM((S // 4,), jnp.float32),
            pltpu.VMEM((S // 4,), jnp.float32),
            pltpu.VMEM((ks[0],), jnp.int32),
            pltpu.VMEM((ks[1],), jnp.int32),
            pltpu.VMEM((ks[2],), jnp.int32),
            pltpu.VMEM((6 * sps[0] * ks[0],), jnp.float32),
            pltpu.VMEM((6 * sps[1] * ks[1],), jnp.float32),
            pltpu.VMEM((6 * sps[2] * ks[2],), jnp.float32),
        ],
    )
    outs = kern(xs.reshape(-1), ys.reshape(-1), zs.reshape(-1),
                cx.reshape(-1), cy.reshape(-1), cz.reshape(-1),
                pc.reshape(-1))
    return list(outs)


def _bq2_body(N, S, ks, r2s, xs_h, ys_h, zs_h, cx_h, cy_h, cz_h, pc_h,
              o1_h, o2_h, o3_h,
              xs_v, ys_v, zs_v, px_v, cxs_v, cys_v, czs_v, pcs_v,
              i1_v, i2_v, i3_v):
    k1, k2, k3 = ks
    r21, r22, r23 = r2s
    w = lax.axis_index("s") * 2 + lax.axis_index("c")
    b = w // 4
    q = w % 4
    S_sub = S // 4
    s0 = q * S_sub
    pltpu.sync_copy(xs_h.at[pl.ds(b * N, N)], xs_v)
    pltpu.sync_copy(ys_h.at[pl.ds(b * N, N)], ys_v)
    pltpu.sync_copy(zs_h.at[pl.ds(b * N, N)], zs_v)
    pltpu.sync_copy(cx_h.at[pl.ds(b * S + s0, S_sub)], cxs_v)
    pltpu.sync_copy(cy_h.at[pl.ds(b * S + s0, S_sub)], cys_v)
    pltpu.sync_copy(cz_h.at[pl.ds(b * S + s0, S_sub)], czs_v)
    pltpu.sync_copy(pc_h.at[pl.ds(b * S + s0, S_sub)], pcs_v)
    iota = lax.iota(jnp.int32, 16)

    def px_step(t, _):
        sl = pl.ds(t * 16, 16)
        xv = xs_v[sl]
        yv = ys_v[sl]
        zv = zs_v[sl]
        px_v[sl] = (xv * xv + yv * yv) + zv * zv
        return 0

    lax.fori_loop(0, N // 16, px_step, 0)

    def scal(ref, i):
        chv = ref[pl.ds((i // 16) * 16, 16)]
        return jnp.sum(jnp.where(iota == i % 16, chv,
                                 jnp.zeros((16,), chv.dtype)))

    def row_fn(rl, _):
        cx_s = scal(cxs_v, rl)
        cy_s = scal(cys_v, rl)
        cz_s = scal(czs_v, rl)
        pc_s = scal(pcs_v, rl)

        def chunk(t, cnts):
            c1, c2, c3 = cnts
            sl = pl.ds(t * 16, 16)
            xv = xs_v[sl]
            yv = ys_v[sl]
            zv = zs_v[sl]
            pxv = px_v[sl]
            dot = (cx_s * xv + cy_s * yv) + cz_s * zv
            d = (pc_s + pxv) - 2.0 * dot
            nvec = t * 16 + iota

            def one(r2, kk, buf, cnt):
                m = d <= r2
                ci = jnp.cumsum(m.astype(jnp.int32))
                pos = cnt + ci - 1
                wr = m & (pos < kk)
                plsc.store_scatter(buf, [jnp.maximum(pos, 0)], nvec,
                                   mask=wr)
                return cnt + jnp.sum(m.astype(jnp.int32))

            c1 = one(r21, k1, i1_v, c1)
            c2 = one(r22, k2, i2_v, c2)
            c3 = one(r23, k3, i3_v, c3)
            return (c1, c2, c3)

        zero = jnp.array(0, jnp.int32)
        c1, c2, c3 = lax.fori_loop(0, N // 16, chunk, (zero, zero, zero))
        sg = s0 + rl

        def emit(kk, cnt, buf, out_h):
            first = scal(buf, 0)

            def fill(ch, _):
                slot = ch * 16 + iota
                cur = buf[pl.ds(ch * 16, 16)]
                buf[pl.ds(ch * 16, 16)] = jnp.where(slot < cnt, cur, first)
                return 0

            lax.fori_loop(0, kk // 16, fill, 0)
            pltpu.sync_copy(buf, out_h.at[pl.ds((b * S + sg) * kk, kk)])

        emit(k1, c1, i1_v, o1_h)
        emit(k2, c2, i2_v, o2_h)
        emit(k3, c3, i3_v, o3_h)
        return 0

    lax.fori_loop(0, S_sub, row_fn, 0)


def _ball_query2(xs, ys, zs, cx, cy, cz, pc, ks, radii):
    B, N = xs.shape
    S = cx.shape[1]
    r2s = tuple(float(r) * float(r) for r in radii)
    kern = pl.kernel(
        functools.partial(_bq2_body, N, S, ks, r2s),
        out_type=[jax.ShapeDtypeStruct((B * S * k,), jnp.int32) for k in ks],
        mesh=plsc.VectorSubcoreMesh(**_SC_MESH),
        compiler_params=pltpu.CompilerParams(needs_layout_passes=False),
        scratch_types=[
            pltpu.VMEM((N,), jnp.float32), pltpu.VMEM((N,), jnp.float32),
            pltpu.VMEM((N,), jnp.float32), pltpu.VMEM((N,), jnp.float32),
            pltpu.VMEM((S // 4,), jnp.float32),
            pltpu.VMEM((S // 4,), jnp.float32),
            pltpu.VMEM((S // 4,), jnp.float32),
            pltpu.VMEM((S // 4,), jnp.float32),
            pltpu.VMEM((ks[0],), jnp.int32),
            pltpu.VMEM((ks[1],), jnp.int32),
            pltpu.VMEM((ks[2],), jnp.int32),
        ],
    )
    outs = kern(xs.reshape(-1), ys.reshape(-1), zs.reshape(-1),
                cx.reshape(-1), cy.reshape(-1), cz.reshape(-1),
                pc.reshape(-1))
    return [o.reshape(B, S * k) for o, k in zip(outs, ks)]


def _mlp1_body(K, sp, g_ref, w1_ref, b1_ref, w2_ref, b2_ref, w3_ref, b3_ref,
               out_ref):
    rows = sp * K
    g6 = g_ref[...].reshape(6, rows)
    h = jnp.maximum(jnp.dot(w1_ref[...], g6) + b1_ref[...], 0.0)
    h = jnp.maximum(jnp.dot(w2_ref[...], h) + b2_ref[...], 0.0)
    h = jnp.maximum(jnp.dot(w3_ref[...], h) + b3_ref[...], 0.0)
    c3 = h.shape[0]
    ht = jnp.transpose(h)
    out_ref[...] = jnp.max(ht.reshape(sp, K, c3), axis=1)[None]


def _mlp1(g, params, K, sp):
    B = 8
    S = g.shape[0] // (B * K * 6)
    (w1, b1), (w2, b2), (w3, b3) = params
    c3 = w3.shape[0]
    grid = (B, S // sp)
    gv = g.reshape(B, S // sp, 6, sp * K)
    return pl.pallas_call(
        functools.partial(_mlp1_body, K, sp),
        grid=grid,
        in_specs=[
            pl.BlockSpec((1, 1, 6, sp * K), lambda b, i: (b, i, 0, 0)),
            pl.BlockSpec(w1.shape, lambda b, i: (0, 0)),
            pl.BlockSpec((b1.shape[0], 1), lambda b, i: (0, 0)),
            pl.BlockSpec(w2.shape, lambda b, i: (0, 0)),
            pl.BlockSpec((b2.shape[0], 1), lambda b, i: (0, 0)),
            pl.BlockSpec(w3.shape, lambda b, i: (0, 0)),
            pl.BlockSpec((b3.shape[0], 1), lambda b, i: (0, 0)),
        ],
        out_specs=pl.BlockSpec((1, sp, c3), lambda b, i: (b, i, 0)),
        out_shape=jax.ShapeDtypeStruct((B, S, c3), jnp.float32),
    )(gv, w1, b1[:, None], w2, b2[:, None], w3, b3[:, None])


def _pre1_body(f_ref, w_ref, b_ref, out_ref):
    out_ref[...] = (jnp.dot(f_ref[...][0], w_ref[...]) + b_ref[...])[None]


def _pre1(f1, w, b):
    B, S, C = f1.shape
    Co = w.shape[1]
    return pl.pallas_call(
        _pre1_body,
        grid=(B,),
        in_specs=[
            pl.BlockSpec((1, S, C), lambda b: (b, 0, 0)),
            pl.BlockSpec((C, Co), lambda b: (0, 0)),
            pl.BlockSpec((Co,), lambda b: (0,)),
        ],
        out_specs=pl.BlockSpec((1, S, Co), lambda b: (b, 0, 0)),
        out_shape=jax.ShapeDtypeStruct((B, S, Co), jnp.float32),
    )(f1, w, b)


def _mlp2_body(K, sp, S1, idx_ref, pre_ref, xyz_ref, c_ref, w1x_ref,
               w2_ref, b2_ref, w3_ref, b3_ref, out_ref):
    rows = sp * K
    idxr = idx_ref[...].reshape(1, rows)
    gt = (lax.broadcasted_iota(jnp.int32, (S1, rows), 0) == idxr
          ).astype(jnp.float32)
    dn = (((0,), (0,)), ((), ()))
    gpre = lax.dot_general(gt, pre_ref[...][0], dimension_numbers=dn)
    gxyz = lax.dot_general(gt, xyz_ref[...][0], dimension_numbers=dn)
    cc = c_ref[...].reshape(sp, 1, 3)
    dxyz = (gxyz.reshape(sp, K, 3) - cc).reshape(rows, 3)
    h = jnp.maximum(gpre + jnp.dot(dxyz, w1x_ref[...]), 0.0)
    h = jnp.maximum(jnp.dot(h, w2_ref[...]) + b2_ref[...], 0.0)
    h = jnp.maximum(jnp.dot(h, w3_ref[...]) + b3_ref[...], 0.0)
    c3 = h.shape[-1]
    out_ref[...] = jnp.max(h.reshape(sp, K, c3), axis=1)[None]


def _mlp2(idx, pre, xyz1, crows, params, K, sp):
    B, S1, C1 = pre.shape
    S = idx.shape[1] // K
    (w1, _), (w2, b2), (w3, b3) = params
    w1x = w1[:, :3]
    c3 = w3.shape[0]
    grid = (B, S // sp)
    iv = idx.reshape(B, S // sp, 8, sp * K // 8)
    return pl.pallas_call(
        functools.partial(_mlp2_body, K, sp, S1),
        grid=grid,
        in_specs=[
            pl.BlockSpec((1, 1, 8, sp * K // 8), lambda b, i: (b, i, 0, 0)),
            pl.BlockSpec((1, S1, C1), lambda b, i: (b, 0, 0)),
            pl.BlockSpec((1, S1, 3), lambda b, i: (b, 0, 0)),
            pl.BlockSpec((1, sp, 3), lambda b, i: (b, i, 0)),
            pl.BlockSpec((3, w1x.shape[0]), lambda b, i: (0, 0)),
            pl.BlockSpec(w2.T.shape, lambda b, i: (0, 0)),
            pl.BlockSpec(b2.shape, lambda b, i: (0,)),
            pl.BlockSpec(w3.T.shape, lambda b, i: (0, 0)),
            pl.BlockSpec(b3.shape, lambda b, i: (0,)),
        ],
        out_specs=pl.BlockSpec((1, sp, c3), lambda b, i: (b, i, 0)),
        out_shape=jax.ShapeDtypeStruct((B, S, c3), jnp.float32),
    )(iv, pre, xyz1, crows, w1x.T, w2.T, b2, w3.T, b3)


def _sa3_body(g_ref, w1_ref, b1_ref, w2_ref, b2_ref, w3_ref, b3_ref, out_ref):
    g = g_ref[...][0]
    h = jnp.maximum(jnp.dot(g, w1_ref[...]) + b1_ref[...], 0.0)
    h = jnp.maximum(jnp.dot(h, w2_ref[...]) + b2_ref[...], 0.0)
    h = jnp.maximum(jnp.dot(h, w3_ref[...]) + b3_ref[...], 0.0)
    out_ref[...] = jnp.max(h, axis=0).reshape(1, 1, -1)


def _sa3(g, params):
    B, S, C = g.shape
    (w1, b1), (w2, b2), (w3, b3) = params
    c3 = w3.shape[0]
    return pl.pallas_call(
        _sa3_body,
        grid=(B,),
        in_specs=[
            pl.BlockSpec((1, S, C), lambda b: (b, 0, 0)),
            pl.BlockSpec(w1.T.shape, lambda b: (0, 0)),
            pl.BlockSpec(b1.shape, lambda b: (0,)),
            pl.BlockSpec(w2.T.shape, lambda b: (0, 0)),
            pl.BlockSpec(b2.shape, lambda b: (0,)),
            pl.BlockSpec(w3.T.shape, lambda b: (0, 0)),
            pl.BlockSpec(b3.shape, lambda b: (0,)),
        ],
        out_specs=pl.BlockSpec((1, 1, c3), lambda b: (b, 0, 0)),
        out_shape=jax.ShapeDtypeStruct((B, 1, c3), jnp.float32),
    )(g, w1.T, b1, w2.T, b2, w3.T, b3).reshape(B, c3)


def _head_body(f_ref, w1_ref, b1_ref, w2_ref, b2_ref, w3_ref, b3_ref, out_ref):
    f = f_ref[...]
    f = jnp.maximum(jnp.dot(f, w1_ref[...]) + b1_ref[...], 0.0)
    f = jnp.maximum(jnp.dot(f, w2_ref[...]) + b2_ref[...], 0.0)
    f = jnp.dot(f, w3_ref[...]) + b3_ref[...]
    out_ref[...] = jax.nn.log_softmax(f, axis=-1)


def _head(feat, params):
    (w1, b1), (w2, b2), (w3, b3) = params
    return pl.pallas_call(
        _head_body,
        out_shape=jax.ShapeDtypeStruct((feat.shape[0], w3.shape[0]),
                                       jnp.float32),
    )(feat, w1.T, b1, w2.T, b2, w3.T, b3)


def kernel(x, sa1_params, sa2_params, sa3_params, head_params):
    B = x.shape[0]
    xs, ys, zs = x[:, 0, :], x[:, 1, :], x[:, 2, :]

    k1 = (16, 32, 128)
    r1 = (0.1, 0.2, 0.4)
    cx1, cy1, cz1, pc1 = _fps(xs, ys, zs, 512)
    sp1 = (64, 32, 16)
    g1a, g1b, g1c = _ball_group1(xs, ys, zs, cx1, cy1, cz1, pc1, k1, r1, sp1)
    f1a = _mlp1(g1a, sa1_params[0], 16, 64)
    f1b = _mlp1(g1b, sa1_params[1], 32, 32)
    f1c = _mlp1(g1c, sa1_params[2], 128, 16)
    f1 = jnp.concatenate([f1a, f1b, f1c], axis=-1)

    k2 = (32, 64, 128)
    r2 = (0.2, 0.4, 0.8)
    cx2, cy2, cz2, pc2 = _fps(cx1, cy1, cz1, 128)
    i2a, i2b, i2c = _ball_query2(cx1, cy1, cz1, cx2, cy2, cz2, pc2, k2, r2)
    w1f = jnp.concatenate([p[0][0][:, 3:] for p in sa2_params], axis=0)
    b1f = jnp.concatenate([p[0][1] for p in sa2_params], axis=0)
    pre1 = _pre1(f1, w1f.T, b1f)
    xyz1 = jnp.stack([cx1, cy1, cz1], axis=-1)
    xyz2 = jnp.stack([cx2, cy2, cz2], axis=-1)
    offs = [0, 64, 192]
    widths = [64, 128, 128]
    f2s = []
    for i, (idx, K, st) in enumerate(zip([i2a, i2b, i2c], k2, [32, 16, 8])):
        pre_i = lax.slice_in_dim(pre1, offs[i], offs[i] + widths[i], axis=2)
        f2s.append(_mlp2(idx, pre_i, xyz1, xyz2, sa2_params[i], K, st))
    f2 = jnp.concatenate(f2s, axis=-1)

    g3 = jnp.concatenate([xyz2, f2], axis=-1)
    f3 = _sa3(g3, sa3_params)
    return _head(f3, head_params)
